# baseline (speedup 1.0000x reference)
import functools

import jax
import jax.numpy as jnp
from jax import lax
from jax.experimental import pallas as pl
from jax.experimental.pallas import tpu as pltpu
from jax.experimental.pallas import tpu_sc as plsc

_B, _L, _E, _W = 8, 2048, 768, 1024
_BL = _B * _L
_BW = _B * _W
_EC = 128
_NCH = 3
_RT = 1024
_SB = 128
_NSB = _RT // _SB
_NG = _NCH * _NSB
_RO = _BW // 16
_ZR = 64

_mesh = plsc.VectorSubcoreMesh(core_axis_name="c", subcore_axis_name="s")


@functools.partial(
    pl.kernel,
    out_type=jax.ShapeDtypeStruct((_BW, _E), jnp.float32),
    mesh=_mesh,
    scratch_types=[
        pltpu.VMEM((_NSB, _SB), jnp.int32),
        pltpu.VMEM((_RT + 16,), jnp.float32),
        pltpu.VMEM((_RT,), jnp.float32),
        pltpu.VMEM((_SB, _EC), jnp.float32),
        pltpu.VMEM((_SB, _EC), jnp.float32),
        pltpu.VMEM((_SB, _EC), jnp.float32),
        pltpu.VMEM((_ZR, _EC), jnp.float32),
        pltpu.VMEM_SHARED((_BW,), jnp.float32),
        pltpu.VMEM_SHARED((_BW, _EC), jnp.float32),
        pltpu.SemaphoreType.DMA,
        pltpu.SemaphoreType.DMA,
        pltpu.SemaphoreType.DMA,
        pltpu.SemaphoreType.DMA,
        pltpu.SemaphoreType.DMA,
        pltpu.SemaphoreType.DMA,
    ],
)
def _sc_pool(seg_hbm, emb_hbm, out_hbm, fid2, invs, small1d, buf0, buf1, buf2,
             ztab, cnt_sh, tab_sh, sem0, sem1, sem2, semw0, semw1, semw2):
    s = lax.axis_index("s")
    c = lax.axis_index("c")
    row0 = s * _RT
    bW = (s // 2) * _W
    own = s * _RO

    zero16 = jnp.zeros((16,), jnp.float32)
    one16 = jnp.ones((16,), jnp.float32)
    bufs = (buf0, buf1, buf2)
    sems = (sem0, sem1, sem2)
    wsems = (semw0, semw1, semw2)

    def _load(g, j):
        k, sb = divmod(g, _NSB)
        e0 = (c * _NCH + k) * _EC
        return pltpu.async_copy(
            emb_hbm.at[pl.ds(row0 + sb * _SB, _SB), pl.ds(e0, _EC)],
            bufs[j], sems[j])

    ld = [_load(0, 0), _load(1, 1), None]

    def _fillz(r, carry):
        for j in range(_EC // 16):
            ztab[r, pl.ds(16 * j, 16)] = zero16
        return carry

    lax.fori_loop(0, _ZR, _fillz, 0)

    def _fillz1(g, carry):
        small1d[pl.ds(16 * g, 16)] = zero16
        return carry

    lax.fori_loop(0, _RO // 16, _fillz1, 0)

    pltpu.sync_copy(seg_hbm.at[pl.ds(s * _NSB, _NSB)], fid2)

    def _addb(r, carry):
        for j in range(_SB // 16):
            fid2[r, pl.ds(16 * j, 16)] = fid2[r, pl.ds(16 * j, 16)] + bW
        return carry

    lax.fori_loop(0, _NSB, _addb, 0)

    zi = [pltpu.async_copy(ztab, tab_sh.at[pl.ds(own + i * _ZR, _ZR)],
                           wsems[i % 3])
          for i in range(_RO // _ZR)]
    pltpu.sync_copy(small1d.at[pl.ds(0, _RO)], cnt_sh.at[pl.ds(own, _RO)])
    for d in zi:
        d.wait()

    def _fillo(g, carry):
        small1d[pl.ds(16 * g, 16)] = one16
        return carry

    lax.fori_loop(0, _SB // 16, _fillo, 0)
    plsc.subcore_barrier()

    cd = [pltpu.async_copy(small1d.at[pl.ds(0, _SB)],
                           cnt_sh.at[fid2.at[sb]], wsems[sb % 3], add=True)
          for sb in range(_NSB)]
    for d in cd:
        d.wait()
    plsc.subcore_barrier()

    gd = [pltpu.async_copy(cnt_sh.at[fid2.at[sb]],
                           small1d.at[pl.ds(sb * _SB, _SB)], wsems[sb % 3])
          for sb in range(_NSB)]
    for d in gd:
        d.wait()

    def _binv(g, carry):
        v = small1d[pl.ds(16 * g, 16)]
        invs[pl.ds(16 * g, 16)] = 1.0 / v
        return carry

    lax.fori_loop(0, _RT // 16, _binv, 0)

    sc = [None, None, None]
    for g in range(_NG):
        k, sb = divmod(g, _NSB)
        j = g % 3
        ld[j].wait()
        buf = bufs[j]

        def _scale(t, carry, sb=sb, buf=buf):
            for u in range(2):
                r = 2 * t + u
                cs = invs[pl.ds(sb * _SB + r, 16)][0]
                for jj in range(_EC // 16):
                    buf[r, pl.ds(16 * jj, 16)] = (
                        buf[r, pl.ds(16 * jj, 16)] * cs)
            return carry

        lax.fori_loop(0, _SB // 2, _scale, 0)
        sc[j] = pltpu.async_copy(buf, tab_sh.at[fid2.at[sb]], sems[j],
                                 add=True)
        nxt = g + 2
        if nxt < _NG:
            jj = nxt % 3
            if sc[jj] is not None:
                sc[jj].wait()
                sc[jj] = None
            ld[jj] = _load(nxt, jj)
        if sb == _NSB - 1:
            for jx in (0, 1, 2):
                if sc[jx] is not None:
                    sc[jx].wait()
                    sc[jx] = None
            plsc.subcore_barrier()
            e0 = (c * _NCH + k) * _EC
            wo = [pltpu.async_copy(
                tab_sh.at[pl.ds(own + i * _ZR, _ZR)],
                out_hbm.at[pl.ds(own + i * _ZR, _ZR), pl.ds(e0, _EC)],
                wsems[i % 3])
                for i in range(_RO // _ZR)]
            for d in wo:
                d.wait()
            if k + 1 < _NCH:
                wz = [pltpu.async_copy(
                    ztab, tab_sh.at[pl.ds(own + i * _ZR, _ZR)],
                    wsems[i % 3])
                    for i in range(_RO // _ZR)]
                for d in wz:
                    d.wait()
                plsc.subcore_barrier()


def kernel(subword_embs, segment_ids):
    seg2 = segment_ids.reshape(_SB, _SB).astype(jnp.int32)
    emb2 = subword_embs.reshape(_BL, _E)
    out = _sc_pool(seg2, emb2)
    return out.reshape(_B, _W, _E)
